# baseline (speedup 1.0000x reference)
import functools

import jax
import jax.numpy as jnp
from jax.experimental import pallas as pl
from jax.experimental.pallas import tpu as pltpu


def _fwd_kernel(x_ref, dist_ref, tw1t_ref, tb1_ref, tw2t_ref, tb2_ref,
                iw_ref, ib_ref, mv_w_ref, mv_b_ref, gate_ref,
                sw0_ref, sb0_ref, gwt_ref, gb_ref, sw3_ref, sb3_ref,
                fw_ref, fb_ref, lng_ref, lnb_ref, pwt_ref, pb_ref,
                fwgt_ref, out_ref, *, heads):
    f32 = jnp.float32
    dot = functools.partial(jnp.dot, preferred_element_type=f32)
    Bx = x_ref.shape[0]
    Nx = dist_ref.shape[0]
    Hx = iw_ref.shape[1]

    dist = dist_ref[...]

    leea_c = (jnp.sum(mv_w_ref[...], axis=0, keepdims=True) + mv_b_ref[...]).T
    sg = jax.nn.sigmoid(gate_ref[0, 0])
    a = jax.nn.sigmoid(fwgt_ref[0, 0])
    b2 = jax.nn.sigmoid(fwgt_ref[0, 1])
    alpha = a / (a + b2)
    beta_w = 1.0 - alpha

    tw1_t = tw1t_ref[...]
    tw2_t = tw2t_ref[...]
    gw_t = gwt_ref[...].reshape(Hx, heads, Hx).sum(axis=1)
    pw_t = pwt_ref[...]
    iw_t = iw_ref[...].T
    sw0_t = sw0_ref[...].T
    sw3_t = sw3_ref[...].T
    fw_t = fw_ref[...].T
    tb1_c = tb1_ref[...].T
    ib_c = ib_ref[...].T
    sb0_c = sb0_ref[...].T
    gb_c = gb_ref[...].T
    sb3_c = sb3_ref[...].T
    fb_c = fb_ref[...].T
    lng_c = lng_ref[...].T
    lnb_c = lnb_ref[...].T
    pb_c = pb_ref[...].T

    ones_1n = jnp.ones((1, Nx), f32)
    ones_1h = jnp.ones((1, Hx), f32)
    inv_n = 1.0 / Nx
    inv_h = 1.0 / Hx

    for tt in range(x_ref.shape[1]):
      xbs = [x_ref[b, tt] for b in range(Bx)]

      x_agg = jnp.concatenate([dot(xb, ones_1n.T) for xb in xbs], axis=1) * inv_n
      h = jnp.maximum(dot(tw1_t, x_agg) + tb1_c, 0.0)
      thr = jax.nn.sigmoid(dot(tw2_t, h) + tb2_ref[0, 0])

      ges = [jnp.where(dist >= thr[0, b], 1.0, 0.0) for b in range(Bx)]
      fracs = [dot(ones_1n, ge) * inv_n for ge in ges]

      cond = jnp.sum(fracs[0]) > 0.0

      chunk = 8
      for b0 in range(0, Bx, chunk):
        bs = range(b0, min(b0 + chunk, Bx))
        imps = [jnp.maximum(dot(iw_t, xbs[b]) + ib_c, 0.0) + sg * leea_c
                for b in bs]
        s0s = [(dot(sw0_t, xbs[b]) + sb0_c) * fracs[b] for b in bs]
        moes = [dot(gw_t, s0) + gb_c for s0 in s0s]
        s1s = [jnp.maximum(jnp.where(cond, moe, s0), 0.0)
               for moe, s0 in zip(moes, s0s)]
        sims = [dot(sw3_t, s1) + sb3_c for s1 in s1s]

        combineds = [alpha * imp + beta_w * sim for imp, sim in zip(imps, sims)]

        fgls = [dot(fw_t[:, :Hx], imp) + dot(fw_t[:, Hx:], sim) + fb_c
                for imp, sim in zip(imps, sims)]
        ms = [dot(ones_1h, fgl) * inv_h for fgl in fgls]
        cs = [fgl - m for fgl, m in zip(fgls, ms)]
        vs = [dot(ones_1h, c * c) * inv_h for c in cs]
        fgs = [0.5 * jnp.tanh((c * jax.lax.rsqrt(v + 1e-5) * lng_c + lnb_c) * 0.5)
               + 0.5 for c, v in zip(cs, vs)]

        for i, b in enumerate(bs):
            z = fgs[i] * (combineds[i] + 1.0 - fgs[i])
            out_ref[b, tt] = dot(pw_t, z) + pb_c


def kernel(x, distances, tw1, tb1, tw2, tb2, iw, ib, mk_w, mk_b, mv_w, mv_b,
           gate, sw0, sb0, gw, gb, sw3, sb3, fw, fb, ln_g, ln_b, pw, pb,
           fusion_weight):
    B, T, N, D = x.shape
    H = iw.shape[1]
    heads = gw.shape[0] // H

    x_t = x.transpose(0, 1, 3, 2)

    row = lambda v: v.reshape(1, -1)
    full = lambda arr: pl.BlockSpec(arr.shape, lambda t: (0,) * arr.ndim)

    operands = (
        x_t, distances, tw1.T, row(tb1), tw2.T, row(tb2), iw, row(ib),
        mv_w, row(mv_b), gate.reshape(1, 1), sw0, row(sb0), gw.T, row(gb),
        sw3, row(sb3), fw, row(fb), row(ln_g), row(ln_b), pw.T, row(pb),
        fusion_weight.reshape(1, 2),
    )
    TCH = 2
    in_specs = [pl.BlockSpec((B, TCH, D, N), lambda t: (0, t, 0, 0))]
    in_specs += [full(op) for op in operands[1:]]

    out = pl.pallas_call(
        functools.partial(_fwd_kernel, heads=heads),
        grid=(T // TCH,),
        in_specs=in_specs,
        out_specs=pl.BlockSpec((B, TCH, D, N), lambda t: (0, t, 0, 0)),
        out_shape=jax.ShapeDtypeStruct((B, T, D, N), x.dtype),
    )(*operands)
    return out.transpose(0, 1, 3, 2)


# speedup vs baseline: 1.0205x; 1.0205x over previous
import functools

import jax
import jax.numpy as jnp
from jax.experimental import pallas as pl
from jax.experimental.pallas import tpu as pltpu


def _fwd_kernel(x_ref, dist_ref, tw1t_ref, tb1_ref, tw2t_ref, tb2_ref,
                iw_ref, ib_ref, mv_w_ref, mv_b_ref, gate_ref,
                sw0_ref, sb0_ref, gwt_ref, gb_ref, sw3_ref, sb3_ref,
                fw_ref, fb_ref, lng_ref, lnb_ref, pwt_ref, pb_ref,
                fwgt_ref, out_ref, *, heads):
    f32 = jnp.float32
    dot = functools.partial(jnp.dot, preferred_element_type=f32)
    Bx = x_ref.shape[0]
    Nx = dist_ref.shape[0]
    Hx = iw_ref.shape[1]

    dist = dist_ref[...]

    leea_c = (jnp.sum(mv_w_ref[...], axis=0, keepdims=True) + mv_b_ref[...]).T
    sg = jax.nn.sigmoid(gate_ref[0, 0])
    a = jax.nn.sigmoid(fwgt_ref[0, 0])
    b2 = jax.nn.sigmoid(fwgt_ref[0, 1])
    alpha = a / (a + b2)
    beta_w = 1.0 - alpha

    tw1_t = tw1t_ref[...]
    tw2_t = tw2t_ref[...]
    gw_t = gwt_ref[...].reshape(Hx, heads, Hx).sum(axis=1)
    pw_t = pwt_ref[...]
    iw_t = iw_ref[...].T
    sw0_t = sw0_ref[...].T
    sw3_t = sw3_ref[...].T
    fw_t = fw_ref[...].T
    tb1_c = tb1_ref[...].T
    ib_c = ib_ref[...].T
    sb0_c = sb0_ref[...].T
    gb_c = gb_ref[...].T
    sb3_c = sb3_ref[...].T
    fb_c = fb_ref[...].T
    lng_c = lng_ref[...].T
    lnb_c = lnb_ref[...].T
    pb_c = pb_ref[...].T

    ones_1n = jnp.ones((1, Nx), f32)
    ones_1h = jnp.ones((1, Hx), f32)
    inv_n = 1.0 / Nx
    inv_h = 1.0 / Hx

    xbs = [x_ref[b, 0] for b in range(Bx)]

    x_agg = jnp.concatenate([dot(xb, ones_1n.T) for xb in xbs], axis=1) * inv_n
    h = jnp.maximum(dot(tw1_t, x_agg) + tb1_c, 0.0)
    thr = jax.nn.sigmoid(dot(tw2_t, h) + tb2_ref[0, 0])

    ges = [jnp.where(dist >= thr[0, b], 1.0, 0.0) for b in range(Bx)]
    fracs = [dot(ones_1n, ge) * inv_n for ge in ges]

    cond = jnp.sum(fracs[0]) > 0.0

    chunk = 8
    for b0 in range(0, Bx, chunk):
        bs = range(b0, min(b0 + chunk, Bx))
        imps = [jnp.maximum(dot(iw_t, xbs[b]) + ib_c, 0.0) + sg * leea_c
                for b in bs]
        s0s = [(dot(sw0_t, xbs[b]) + sb0_c) * fracs[b] for b in bs]
        moes = [dot(gw_t, s0) + gb_c for s0 in s0s]
        s1s = [jnp.maximum(jnp.where(cond, moe, s0), 0.0)
               for moe, s0 in zip(moes, s0s)]
        sims = [dot(sw3_t, s1) + sb3_c for s1 in s1s]

        combineds = [alpha * imp + beta_w * sim for imp, sim in zip(imps, sims)]

        fgls = [dot(fw_t[:, :Hx], imp) + dot(fw_t[:, Hx:], sim) + fb_c
                for imp, sim in zip(imps, sims)]
        ms = [dot(ones_1h, fgl) * inv_h for fgl in fgls]
        cs = [fgl - m for fgl, m in zip(fgls, ms)]
        vs = [dot(ones_1h, c * c) * inv_h for c in cs]
        fgs = [0.5 * jnp.tanh((c * jax.lax.rsqrt(v + 1e-5) * lng_c + lnb_c) * 0.5)
               + 0.5 for c, v in zip(cs, vs)]

        for i, b in enumerate(bs):
            z = fgs[i] * (combineds[i] + 1.0 - fgs[i])
            out_ref[b, 0] = dot(pw_t, z) + pb_c


def kernel(x, distances, tw1, tb1, tw2, tb2, iw, ib, mk_w, mk_b, mv_w, mv_b,
           gate, sw0, sb0, gw, gb, sw3, sb3, fw, fb, ln_g, ln_b, pw, pb,
           fusion_weight):
    B, T, N, D = x.shape
    H = iw.shape[1]
    heads = gw.shape[0] // H

    x_t = x.transpose(0, 1, 3, 2)

    row = lambda v: v.reshape(1, -1)
    full = lambda arr: pl.BlockSpec(arr.shape, lambda t: (0,) * arr.ndim)

    operands = (
        x_t, distances, tw1.T, row(tb1), tw2.T, row(tb2), iw, row(ib),
        mv_w, row(mv_b), gate.reshape(1, 1), sw0, row(sb0), gw.T, row(gb),
        sw3, row(sb3), fw, row(fb), row(ln_g), row(ln_b), pw.T, row(pb),
        fusion_weight.reshape(1, 2),
    )
    in_specs = [pl.BlockSpec((B, 1, D, N), lambda t: (0, t, 0, 0))]
    in_specs += [full(op) for op in operands[1:]]

    out = pl.pallas_call(
        functools.partial(_fwd_kernel, heads=heads),
        grid=(T,),
        in_specs=in_specs,
        out_specs=pl.BlockSpec((B, 1, D, N), lambda t: (0, t, 0, 0)),
        out_shape=jax.ShapeDtypeStruct((B, T, D, N), x.dtype),
    )(*operands)
    return out.transpose(0, 1, 3, 2)
